# baseline (speedup 1.0000x reference)
import functools

import jax
import jax.numpy as jnp
from jax import lax
from jax.experimental import pallas as pl
from jax.experimental.pallas import tpu as pltpu
from jax.experimental.pallas import tpu_sc as plsc

_D = _H = _W = 128
_HW = _H * _W
_DHW = _D * _HW
_C = 16
_L = 16
_CH = 128
_NC = 2
_NS = 16
_NW = _NC * _NS
_OFFS = (0, 1, _W, _W + 1, _HW, _HW + 1, _HW + _W, _HW + _W + 1)


def _axis_coords(v, dim):
    t = (v - 0.5) * 2.0
    i = (t + 1.0) * 0.5 * jnp.float32(dim - 1)
    i0 = jnp.minimum(i.astype(jnp.int32), dim - 2)
    w1 = i - i0.astype(jnp.float32)
    return i0, w1, 1.0 - w1


def _weights(vx, vy, vz):
    x0, wx1, wx0 = _axis_coords(vx, _W)
    y0, wy1, wy0 = _axis_coords(vy, _H)
    z0, wz1, wz0 = _axis_coords(vz, _D)
    a00 = wz0 * wy0
    a01 = wz0 * wy1
    a10 = wz1 * wy0
    a11 = wz1 * wy1
    ws = (a00 * wx0, a00 * wx1, a01 * wx0, a01 * wx1,
          a10 * wx0, a10 * wx1, a11 * wx0, a11 * wx1)
    return (x0, y0, z0), ws


def _sc_body(table, cx, cy, cz, out, *scr):
    cbuf = (scr[0:3], scr[3:6])
    idx = (scr[6:14], scr[14:22])
    rows = (scr[22:30], scr[30:38])
    obuf = scr[38:40]
    sem_g, sem_c, sem_o = scr[40:43]

    nper = cx.shape[0] // _NW
    chunks = nper // _CH
    wid = lax.axis_index("s") * _NC + lax.axis_index("c")
    cin = (cx, cy, cz)

    def base_of(s):
        return wid * nper + s * _CH

    def start_coords(s, b):
        return [pltpu.async_copy(cin[k].at[pl.ds(base_of(s), _CH)],
                                 cbuf[b][k], sem_c) for k in range(3)]

    def wait_coords(b):
        for k in range(3):
            pltpu.make_async_copy(cin[k].at[pl.ds(0, _CH)],
                                  cbuf[b][k], sem_c).wait()

    def compute_idx(b):
        gx, gy, gz = cbuf[b]
        for g in range(_CH // _L):
            sl = pl.ds(g * _L, _L)
            (x0, y0, z0), _ = _weights(gx[sl], gy[sl], gz[sl])
            v = z0 * _HW + y0 * _W + x0
            for j in range(8):
                idx[b][j][sl] = v + _OFFS[j]

    def fire_gathers(b):
        for j in range(8):
            pltpu.async_copy(table.at[idx[b][j]], rows[b][j], sem_g)

    def wait_gathers(b):
        for j in range(8):
            pltpu.make_async_copy(table.at[idx[b][j]], rows[b][j],
                                  sem_g).wait()

    def blend(b):
        gx, gy, gz = cbuf[b]
        rws = rows[b]
        ob = obuf[b]

        def group(g, c):
            sl = pl.ds(g * _L, _L)
            _, ws = _weights(gx[sl], gy[sl], gz[sl])
            for l in range(_L):
                p = g * _L + l
                acc = ws[0][l] * rws[0][p, :]
                for j in range(1, 8):
                    acc = acc + ws[j][l] * rws[j][p, :]
                ob[p, :] = acc
            return c

        lax.fori_loop(0, _CH // _L, group, 0)

    def wait_out(b):
        pltpu.make_async_copy(obuf[b], out.at[pl.ds(0, _CH)], sem_o).wait()

    for cp in start_coords(0, 0):
        cp.wait()
    compute_idx(0)
    fire_gathers(0)
    start_coords(1, 1)

    def body(s, carry):
        b = lax.rem(s, 2)

        def side(bb):
            nb = 1 - bb
            wait_coords(nb)
            compute_idx(nb)
            wait_gathers(bb)
            fire_gathers(nb)

            @pl.when(s >= 2)
            def _():
                wait_out(bb)

            blend(bb)
            pltpu.async_copy(obuf[bb], out.at[pl.ds(base_of(s), _CH)], sem_o)

            @pl.when(s < chunks - 2)
            def _():
                start_coords(s + 2, bb)

        @pl.when(b == 0)
        def _():
            side(0)

        @pl.when(b == 1)
        def _():
            side(1)

        return carry

    lax.fori_loop(0, chunks - 1, body, 0)

    lastb = (chunks - 1) % 2
    wait_gathers(lastb)
    wait_out(lastb)
    blend(lastb)
    wait_out(1 - lastb)
    pltpu.sync_copy(obuf[lastb], out.at[pl.ds(base_of(chunks - 1), _CH)])


def kernel(x, data):
    n = x.shape[0]
    eye_c = jnp.eye(_C, dtype=jnp.float32)
    table = jax.lax.dot_general(
        data[0].reshape(_C, _DHW), eye_c,
        dimension_numbers=(((0,), (0,)), ((), ())),
        preferred_element_type=jnp.float32,
    )
    e2 = jnp.zeros((3,), jnp.float32).at[2].set(1.0)
    e1 = jnp.zeros((3,), jnp.float32).at[1].set(1.0)
    e0 = jnp.zeros((3,), jnp.float32).at[0].set(1.0)
    cx = x @ e2
    cy = x @ e1
    cz = x @ e0

    mesh = plsc.VectorSubcoreMesh(core_axis_name="c", subcore_axis_name="s")
    scratch = (
        [pltpu.VMEM((_CH,), jnp.float32) for _ in range(6)]
        + [pltpu.VMEM((_CH,), jnp.int32) for _ in range(16)]
        + [pltpu.VMEM((_CH, _C), jnp.float32) for _ in range(16)]
        + [pltpu.VMEM((_CH, _C), jnp.float32) for _ in range(2)]
        + [pltpu.SemaphoreType.DMA for _ in range(3)]
    )
    run = functools.partial(
        pl.kernel,
        out_type=jax.ShapeDtypeStruct((n, _C), jnp.float32),
        mesh=mesh,
        scratch_types=scratch,
        compiler_params=pltpu.CompilerParams(use_tc_tiling_on_sc=False),
    )(_sc_body)
    return run(table, cx, cy, cz)


# speedup vs baseline: 2.6464x; 2.6464x over previous
import functools

import jax
import jax.numpy as jnp
from jax import lax
from jax.experimental import pallas as pl
from jax.experimental.pallas import tpu as pltpu
from jax.experimental.pallas import tpu_sc as plsc

_D = _H = _W = 128
_HW = _H * _W
_DHW = _D * _HW
_C = 16
_L = 16
_CH = 128
_NC = 2
_NS = 16
_NW = _NC * _NS
_OFFS = (0, 1, _W, _W + 1, _HW, _HW + 1, _HW + _W, _HW + _W + 1)


def _axis_coords(v, dim):
    t = (v - 0.5) * 2.0
    i = (t + 1.0) * 0.5 * jnp.float32(dim - 1)
    i0 = jnp.minimum(i.astype(jnp.int32), dim - 2)
    w1 = i - i0.astype(jnp.float32)
    return i0, w1, 1.0 - w1


def _weights(vx, vy, vz):
    x0, wx1, wx0 = _axis_coords(vx, _W)
    y0, wy1, wy0 = _axis_coords(vy, _H)
    z0, wz1, wz0 = _axis_coords(vz, _D)
    a00 = wz0 * wy0
    a01 = wz0 * wy1
    a10 = wz1 * wy0
    a11 = wz1 * wy1
    ws = (a00 * wx0, a00 * wx1, a01 * wx0, a01 * wx1,
          a10 * wx0, a10 * wx1, a11 * wx0, a11 * wx1)
    return (x0, y0, z0), ws


def _sc_body(table, cx, cy, cz, out, *scr):
    cbuf = (scr[0:3], scr[3:6])
    idx = (scr[6:14], scr[14:22])
    rows = (scr[22:30], scr[30:38])
    obuf = scr[38:40]
    sem_g, sem_c, sem_o = scr[40:43]

    nper = cx.shape[0] // _NW
    chunks = nper // _CH
    wid = lax.axis_index("s") * _NC + lax.axis_index("c")
    cin = (cx, cy, cz)

    def base_of(s):
        return wid * nper + s * _CH

    def start_coords(s, b):
        return [pltpu.async_copy(cin[k].at[pl.ds(base_of(s), _CH)],
                                 cbuf[b][k], sem_c) for k in range(3)]

    def wait_coords(b):
        for k in range(3):
            pltpu.make_async_copy(cin[k].at[pl.ds(0, _CH)],
                                  cbuf[b][k], sem_c).wait()

    def compute_idx(b):
        gx, gy, gz = cbuf[b]
        for g in range(_CH // _L):
            sl = pl.ds(g * _L, _L)
            (x0, y0, z0), _ = _weights(gx[sl], gy[sl], gz[sl])
            v = z0 * _HW + y0 * _W + x0
            for j in range(8):
                idx[b][j][sl] = v + _OFFS[j]

    def fire_gathers(b):
        for j in range(8):
            pltpu.async_copy(table.at[idx[b][j]], rows[b][j], sem_g)

    def wait_gathers(b):
        for j in range(8):
            pltpu.make_async_copy(table.at[idx[b][j]], rows[b][j],
                                  sem_g).wait()

    def blend(b):
        gx, gy, gz = cbuf[b]
        rws = rows[b]
        ob = obuf[b]

        def group(g, c):
            sl = pl.ds(g * _L, _L)
            _, ws = _weights(gx[sl], gy[sl], gz[sl])
            for l in range(_L):
                p = g * _L + l
                acc = ws[0][l] * rws[0][p, :]
                for j in range(1, 8):
                    acc = acc + ws[j][l] * rws[j][p, :]
                ob[p, :] = acc
            return c

        lax.fori_loop(0, _CH // _L, group, 0)

    def wait_out(b):
        pltpu.make_async_copy(obuf[b], out.at[pl.ds(0, _CH)], sem_o).wait()

    for cp in start_coords(0, 0):
        cp.wait()
    compute_idx(0)
    fire_gathers(0)
    start_coords(1, 1)

    def body(s, carry):
        b = lax.rem(s, 2)

        def side(bb):
            nb = 1 - bb
            wait_coords(nb)
            compute_idx(nb)
            wait_gathers(bb)
            fire_gathers(nb)

            @pl.when(s >= 2)
            def _():
                wait_out(bb)

            blend(bb)
            pltpu.async_copy(obuf[bb], out.at[pl.ds(base_of(s), _CH)], sem_o)

            @pl.when(s < chunks - 2)
            def _():
                start_coords(s + 2, bb)

        @pl.when(b == 0)
        def _():
            side(0)

        @pl.when(b == 1)
        def _():
            side(1)

        return carry

    lax.fori_loop(0, chunks - 1, body, 0)

    lastb = (chunks - 1) % 2
    wait_gathers(lastb)
    wait_out(lastb)
    blend(lastb)
    wait_out(1 - lastb)
    pltpu.sync_copy(obuf[lastb], out.at[pl.ds(base_of(chunks - 1), _CH)])


def kernel(x, data):
    n = x.shape[0]
    table = jnp.transpose(data[0], (1, 2, 3, 0)).reshape(_DHW, _C)
    e2 = jnp.zeros((3,), jnp.float32).at[2].set(1.0)
    e1 = jnp.zeros((3,), jnp.float32).at[1].set(1.0)
    e0 = jnp.zeros((3,), jnp.float32).at[0].set(1.0)
    cx = x @ e2
    cy = x @ e1
    cz = x @ e0

    mesh = plsc.VectorSubcoreMesh(core_axis_name="c", subcore_axis_name="s")
    scratch = (
        [pltpu.VMEM((_CH,), jnp.float32) for _ in range(6)]
        + [pltpu.VMEM((_CH,), jnp.int32) for _ in range(16)]
        + [pltpu.VMEM((_CH, _C), jnp.float32) for _ in range(16)]
        + [pltpu.VMEM((_CH, _C), jnp.float32) for _ in range(2)]
        + [pltpu.SemaphoreType.DMA for _ in range(3)]
    )
    run = functools.partial(
        pl.kernel,
        out_type=jax.ShapeDtypeStruct((n, _C), jnp.float32),
        mesh=mesh,
        scratch_types=scratch,
        compiler_params=pltpu.CompilerParams(use_tc_tiling_on_sc=False),
    )(_sc_body)
    return run(table, cx, cy, cz)
